# baseline (speedup 1.0000x reference)
import functools

import jax
import jax.numpy as jnp
from jax import lax
from jax.experimental import pallas as pl
from jax.experimental.pallas import tpu as pltpu
from jax.experimental.pallas import tpu_sc as plsc

NN = 10000
EE = 320000
DD = 128
HH = 32
CC = 64
NC = 2
NS = 16
NW = NC * NS
CH = 128
NB = 5
NCH = 160
EPW = NCH * CH
EPAD = NS * EPW
NCHD = EPAD // (NW * CH)
NTRASH = 112
NACC = NN + NTRASH
RPS = NACC // NS
RB = 2000
GRID = NN // RB

_MESH = plsc.VectorSubcoreMesh(
    core_axis_name="c", subcore_axis_name="s", num_cores=NC, num_subcores=NS)


@functools.partial(
    pl.kernel,
    out_type=jax.ShapeDtypeStruct((NACC, DD), jnp.float32),
    mesh=_MESH,
    scratch_types=[
        pltpu.VMEM((NCHD, CH), jnp.int32),
        pltpu.VMEM((CH, 16), jnp.float32),
        pltpu.VMEM_SHARED((NACC, 16), jnp.float32),
        [pltpu.SemaphoreType.DMA] * 4,
    ],
    compiler_params=pltpu.CompilerParams(use_tc_tiling_on_sc=False, disable_bounds_checks=True, disable_semaphore_checks=True),
)
def _deg_kernel(dst_hbm, ones_hbm, zeros_hbm, out_hbm, idx_d, ones_v, acc, sems):
    c = lax.axis_index("c")
    s = lax.axis_index("s")
    wid = s * NC + c
    pltpu.async_copy(dst_hbm.at[wid], idx_d, sems[0])
    pltpu.async_copy(ones_hbm, ones_v, sems[1])
    pltpu.async_copy(zeros_hbm.at[pl.ds(s * RPS, RPS)],
                     acc.at[pl.ds(s * RPS, RPS)], sems[2])
    pltpu.make_async_copy(dst_hbm.at[wid], idx_d, sems[0]).wait()
    pltpu.make_async_copy(ones_hbm, ones_v, sems[1]).wait()
    pltpu.make_async_copy(zeros_hbm.at[pl.ds(s * RPS, RPS)],
                          acc.at[pl.ds(s * RPS, RPS)], sems[2]).wait()
    plsc.subcore_barrier()

    for b in range(4):
        pltpu.async_copy(ones_v, acc.at[idx_d.at[b]], sems[b], add=True)

    def step(i, carry):
        j = i * 4
        for b in range(4):
            pltpu.make_async_copy(ones_v, acc.at[idx_d.at[j + b]],
                                  sems[b]).wait()
            pltpu.async_copy(ones_v, acc.at[idx_d.at[j + 4 + b]], sems[b],
                             add=True)
        return carry

    lax.fori_loop(0, NCHD // 4 - 1, step, 0)
    for b in range(4):
        pltpu.make_async_copy(ones_v, acc.at[idx_d.at[NCHD - 4 + b]],
                              sems[b]).wait()
    plsc.subcore_barrier()
    pltpu.sync_copy(acc.at[pl.ds(s * RPS, RPS)],
                    out_hbm.at[pl.ds(s * RPS, RPS), pl.ds(c * 16, 16)])


def _make_agg(width, nb):
    @functools.partial(
        pl.kernel,
        out_type=jax.ShapeDtypeStruct((NACC, DD), jnp.float32),
        mesh=_MESH,
        scratch_types=[
            pltpu.VMEM((NCH, CH), jnp.int32),
            pltpu.VMEM((NCH, CH), jnp.int32),
            [pltpu.VMEM((CH, width), jnp.float32)] * nb,
            pltpu.VMEM_SHARED((NACC, width), jnp.float32),
            [pltpu.SemaphoreType.DMA] * nb,
            [pltpu.SemaphoreType.DMA] * nb,
        ],
        compiler_params=pltpu.CompilerParams(use_tc_tiling_on_sc=False, disable_bounds_checks=True, disable_semaphore_checks=True),
    )
    def agg(src_hbm, dst_hbm, table_hbm, zeros_hbm, out_hbm,
            idx_s, idx_d, rows, acc, gsem, ssem):
        c = lax.axis_index("c")
        s = lax.axis_index("s")
        pltpu.async_copy(src_hbm.at[s], idx_s, gsem[0])
        pltpu.async_copy(dst_hbm.at[s], idx_d, gsem[1])
        pltpu.async_copy(zeros_hbm.at[pl.ds(s * RPS, RPS)],
                         acc.at[pl.ds(s * RPS, RPS)], gsem[2])
        pltpu.make_async_copy(src_hbm.at[s], idx_s, gsem[0]).wait()
        pltpu.make_async_copy(dst_hbm.at[s], idx_d, gsem[1]).wait()
        pltpu.make_async_copy(zeros_hbm.at[pl.ds(s * RPS, RPS)],
                              acc.at[pl.ds(s * RPS, RPS)], gsem[2]).wait()
        plsc.subcore_barrier()
        table = table_hbm.at[c]

        for b in range(nb):
            pltpu.async_copy(table.at[idx_s.at[b]], rows[b], gsem[b])

        def step(i, carry):
            j = i * nb
            for b in range(nb):
                pltpu.make_async_copy(table.at[idx_s.at[j + b]], rows[b],
                                      gsem[b]).wait()
                pltpu.async_copy(rows[b], acc.at[idx_d.at[j + b]], ssem[b],
                                 add=True)
            for b in range(nb):
                pltpu.make_async_copy(rows[b], acc.at[idx_d.at[j + b]],
                                      ssem[b]).wait()
                pltpu.async_copy(table.at[idx_s.at[j + nb + b]], rows[b],
                                 gsem[b])
            return carry

        lax.fori_loop(0, NCH // nb - 1, step, 0)
        last = NCH - nb
        for b in range(nb):
            pltpu.make_async_copy(table.at[idx_s.at[last + b]], rows[b],
                                  gsem[b]).wait()
            pltpu.sync_copy(rows[b], acc.at[idx_d.at[last + b]], add=True)
        plsc.subcore_barrier()
        pltpu.sync_copy(acc.at[pl.ds(s * RPS, RPS)],
                        out_hbm.at[pl.ds(s * RPS, RPS),
                                   pl.ds(c * width, width)])

    return agg


_agg128 = _make_agg(DD // 2, NB)
_agg64 = _make_agg(CC // 2, 8)


def _dinv(degp_blk):
    deg = degp_blk[:, 0:1] + degp_blk[:, 16:17] + 1.0
    return lax.rsqrt(deg)


def _tc0_body(x_ref, wa, wb, wc, wd, v_ref):
    W = jnp.concatenate([wa[...], wb[...], wc[...], wd[...]], axis=1)
    v_ref[...] = jnp.dot(x_ref[...], W,
                         precision=lax.Precision.HIGHEST,
                         preferred_element_type=jnp.float32)


def _tc0(x, W1a, W1b, W2a, W2b):
    return pl.pallas_call(
        _tc0_body,
        grid=(GRID,),
        in_specs=[
            pl.BlockSpec((RB, DD), lambda i: (i, 0)),
            pl.BlockSpec((DD, HH), lambda i: (0, 0)),
            pl.BlockSpec((DD, HH), lambda i: (0, 0)),
            pl.BlockSpec((DD, HH), lambda i: (0, 0)),
            pl.BlockSpec((DD, HH), lambda i: (0, 0)),
        ],
        out_specs=pl.BlockSpec((RB, DD), lambda i: (i, 0)),
        out_shape=jax.ShapeDtypeStruct((NN, DD), jnp.float32),
    )(x, W1a, W1b, W2a, W2b)


def _tc1_body(x_ref, wa, wb, wc, wd, degp_ref, u_ref):
    dinv = _dinv(degp_ref[...])
    W = jnp.concatenate([wa[...], wb[...], wc[...], wd[...]], axis=1)
    U = dinv * jnp.dot(x_ref[...], W,
                       precision=lax.Precision.HIGHEST,
                       preferred_element_type=jnp.float32)
    u_ref[0] = U[:, 0:DD // 2]
    u_ref[1] = U[:, DD // 2:DD]


def _tc1(x, W1a, W1b, W2a, W2b, degp):
    return pl.pallas_call(
        _tc1_body,
        grid=(GRID,),
        in_specs=[
            pl.BlockSpec((RB, DD), lambda i: (i, 0)),
            pl.BlockSpec((DD, HH), lambda i: (0, 0)),
            pl.BlockSpec((DD, HH), lambda i: (0, 0)),
            pl.BlockSpec((DD, HH), lambda i: (0, 0)),
            pl.BlockSpec((DD, HH), lambda i: (0, 0)),
            pl.BlockSpec((RB, DD), lambda i: (i, 0)),
        ],
        out_specs=pl.BlockSpec((NC, RB, DD // 2), lambda i: (0, i, 0)),
        out_shape=jax.ShapeDtypeStruct((NC, NN, DD // 2), jnp.float32),
    )(x, W1a, W1b, W2a, W2b, degp)


def _tc2_body(sp_ref, u_ref, degp_ref, b1a, b1b, b2a, b2b, wc1, wc2,
              h1_ref, h2_ref, h3_ref, h4_ref, x1_ref, x2_ref, u2_ref):
    dinv = _dinv(degp_ref[...])
    bstack = jnp.concatenate([b1a[...], b1b[...], b2a[...], b2b[...]], axis=1)
    U = jnp.concatenate([u_ref[0], u_ref[1]], axis=1)
    Y = dinv * (sp_ref[...] + U) + bstack
    Yact = jnp.maximum(Y, 0.0)
    h1_ref[...] = Yact[:, 0:HH]
    h2_ref[...] = Yact[:, HH:2 * HH]
    h3_ref[...] = Yact[:, 2 * HH:3 * HH]
    h4_ref[...] = Yact[:, 3 * HH:4 * HH]
    x1 = Yact[:, 0:HH] + Yact[:, HH:2 * HH]
    x2 = Yact[:, 2 * HH:3 * HH] + Yact[:, 3 * HH:4 * HH]
    x1_ref[...] = x1
    x2_ref[...] = x2
    U2 = dinv * jnp.dot(x1 + x2, wc1[...] + wc2[...],
                        precision=lax.Precision.HIGHEST,
                        preferred_element_type=jnp.float32)
    u2_ref[0] = U2[:, 0:CC // 2]
    u2_ref[1] = U2[:, CC // 2:CC]


def _tc2(Sp, U, degp, b1a, b1b, b2a, b2b, Wc1, Wc2):
    hspec = pl.BlockSpec((RB, HH), lambda i: (i, 0))
    hshape = jax.ShapeDtypeStruct((NN, HH), jnp.float32)
    return pl.pallas_call(
        _tc2_body,
        grid=(GRID,),
        in_specs=[
            pl.BlockSpec((RB, DD), lambda i: (i, 0)),
            pl.BlockSpec((NC, RB, DD // 2), lambda i: (0, i, 0)),
            pl.BlockSpec((RB, DD), lambda i: (i, 0)),
            pl.BlockSpec((1, HH), lambda i: (0, 0)),
            pl.BlockSpec((1, HH), lambda i: (0, 0)),
            pl.BlockSpec((1, HH), lambda i: (0, 0)),
            pl.BlockSpec((1, HH), lambda i: (0, 0)),
            pl.BlockSpec((HH, CC), lambda i: (0, 0)),
            pl.BlockSpec((HH, CC), lambda i: (0, 0)),
        ],
        out_specs=[hspec, hspec, hspec, hspec, hspec, hspec,
                   pl.BlockSpec((NC, RB, CC // 2), lambda i: (0, i, 0))],
        out_shape=[hshape, hshape, hshape, hshape, hshape, hshape,
                   jax.ShapeDtypeStruct((NC, NN, CC // 2), jnp.float32)],
    )(Sp, U, degp, b1a, b1b, b2a, b2b, Wc1, Wc2)


def _tc3_body(s2p_ref, u2_ref, degp_ref, bc1, bc2, out_ref):
    dinv = _dinv(degp_ref[...])
    U2 = jnp.concatenate([u2_ref[0], u2_ref[1]], axis=1)
    ctot = dinv * (s2p_ref[:, 0:CC] + U2) + (bc1[...] + bc2[...])
    m = jnp.max(ctot, axis=1, keepdims=True)
    lse = m + jnp.log(jnp.sum(jnp.exp(ctot - m), axis=1, keepdims=True))
    out_ref[...] = ctot - lse


def _tc3(S2p, U2, degp, bc1, bc2):
    return pl.pallas_call(
        _tc3_body,
        grid=(GRID,),
        in_specs=[
            pl.BlockSpec((RB, DD), lambda i: (i, 0)),
            pl.BlockSpec((NC, RB, CC // 2), lambda i: (0, i, 0)),
            pl.BlockSpec((RB, DD), lambda i: (i, 0)),
            pl.BlockSpec((1, CC), lambda i: (0, 0)),
            pl.BlockSpec((1, CC), lambda i: (0, 0)),
        ],
        out_specs=pl.BlockSpec((RB, CC), lambda i: (i, 0)),
        out_shape=jax.ShapeDtypeStruct((NN, CC), jnp.float32),
    )(S2p, U2, degp, bc1, bc2)


def kernel(x, edge_index, train_mask,
           W1a, b1a, W1b, b1b, W2a, b2a, W2b, b2b, Wc1, bc1, Wc2, bc2):
    src0 = edge_index[0]
    dst0 = edge_index[1]
    pad = EPAD - EE
    ar = jnp.arange(pad, dtype=jnp.int32)
    pad_src = (ar * 997) % NN
    pad_dst = NN + (ar % NTRASH)
    src_p = jnp.concatenate([src0, pad_src])
    dst_p = jnp.concatenate([dst0, pad_dst])
    src3 = src_p.reshape(NS, NCH, CH)
    dst3 = dst_p.reshape(NS, NCH, CH)
    dst3w = dst_p.reshape(NW, NCHD, CH)

    ones16 = jnp.ones((CH, 16), jnp.float32)
    zeros16 = jnp.zeros((NACC, 16), jnp.float32)
    zeros64 = jnp.zeros((NACC, DD // 2), jnp.float32)
    zeros32 = jnp.zeros((NACC, CC // 2), jnp.float32)

    degp = _deg_kernel(dst3w, ones16, zeros16)
    U = _tc1(x, W1a, W1b, W2a, W2b, degp)
    Sp = _agg128(src3, dst3, U, zeros64)
    h1, h2, h3, h4, x1, x2, U2 = _tc2(
        Sp, U, degp,
        b1a.reshape(1, HH), b1b.reshape(1, HH),
        b2a.reshape(1, HH), b2b.reshape(1, HH), Wc1, Wc2)
    S2p = _agg64(src3, dst3, U2, zeros32)
    out = _tc3(S2p, U2, degp, bc1.reshape(1, CC), bc2.reshape(1, CC))
    return (out, h1, h2, h3, h4, x1, x2)


# speedup vs baseline: 1.0003x; 1.0003x over previous
import functools

import jax
import jax.numpy as jnp
from jax import lax
from jax.experimental import pallas as pl
from jax.experimental.pallas import tpu as pltpu
from jax.experimental.pallas import tpu_sc as plsc

NN = 10000
EE = 320000
DD = 128
HH = 32
CC = 64
NC = 2
NS = 16
NW = NC * NS
CH = 128
NB = 5
NCH = 160
EPW = NCH * CH
EPAD = NS * EPW
NCHD = EPAD // (NW * CH)
NTRASH = 112
NACC = NN + NTRASH
RPS = NACC // NS
RB = 2000
GRID = NN // RB

_MESH = plsc.VectorSubcoreMesh(
    core_axis_name="c", subcore_axis_name="s", num_cores=NC, num_subcores=NS)


@functools.partial(
    pl.kernel,
    out_type=jax.ShapeDtypeStruct((NACC, DD), jnp.float32),
    mesh=_MESH,
    scratch_types=[
        pltpu.VMEM((NCHD, CH), jnp.int32),
        pltpu.VMEM((CH, 16), jnp.float32),
        pltpu.VMEM_SHARED((NACC, 16), jnp.float32),
        [pltpu.SemaphoreType.DMA] * 4,
    ],
    compiler_params=pltpu.CompilerParams(use_tc_tiling_on_sc=False, disable_bounds_checks=True, disable_semaphore_checks=True),
)
def _deg_kernel(dst_hbm, ones_hbm, zeros_hbm, out_hbm, idx_d, ones_v, acc, sems):
    c = lax.axis_index("c")
    s = lax.axis_index("s")
    wid = s * NC + c
    pltpu.async_copy(dst_hbm.at[wid], idx_d, sems[0])
    pltpu.async_copy(ones_hbm, ones_v, sems[1])
    pltpu.async_copy(zeros_hbm.at[pl.ds(s * RPS, RPS)],
                     acc.at[pl.ds(s * RPS, RPS)], sems[2])
    pltpu.make_async_copy(dst_hbm.at[wid], idx_d, sems[0]).wait()
    pltpu.make_async_copy(ones_hbm, ones_v, sems[1]).wait()
    pltpu.make_async_copy(zeros_hbm.at[pl.ds(s * RPS, RPS)],
                          acc.at[pl.ds(s * RPS, RPS)], sems[2]).wait()
    plsc.subcore_barrier()

    for b in range(4):
        pltpu.async_copy(ones_v, acc.at[idx_d.at[b]], sems[b], add=True)

    def step(i, carry):
        j = i * 4
        for b in range(4):
            pltpu.make_async_copy(ones_v, acc.at[idx_d.at[j + b]],
                                  sems[b]).wait()
            pltpu.async_copy(ones_v, acc.at[idx_d.at[j + 4 + b]], sems[b],
                             add=True)
        return carry

    lax.fori_loop(0, NCHD // 4 - 1, step, 0)
    for b in range(4):
        pltpu.make_async_copy(ones_v, acc.at[idx_d.at[NCHD - 4 + b]],
                              sems[b]).wait()
    plsc.subcore_barrier()
    pltpu.sync_copy(acc.at[pl.ds(s * RPS, RPS)],
                    out_hbm.at[pl.ds(s * RPS, RPS), pl.ds(c * 16, 16)])


def _make_agg(width, nb):
    @functools.partial(
        pl.kernel,
        out_type=jax.ShapeDtypeStruct((NACC, DD), jnp.float32),
        mesh=_MESH,
        scratch_types=[
            pltpu.VMEM((NCH, CH), jnp.int32),
            pltpu.VMEM((NCH, CH), jnp.int32),
            [pltpu.VMEM((CH, width), jnp.float32)] * nb,
            pltpu.VMEM_SHARED((NACC, width), jnp.float32),
            [pltpu.SemaphoreType.DMA] * nb,
            [pltpu.SemaphoreType.DMA] * nb,
        ],
        compiler_params=pltpu.CompilerParams(use_tc_tiling_on_sc=False, disable_bounds_checks=True, disable_semaphore_checks=True),
    )
    def agg(src_hbm, dst_hbm, table_hbm, zeros_hbm, out_hbm,
            idx_s, idx_d, rows, acc, gsem, ssem):
        c = lax.axis_index("c")
        s = lax.axis_index("s")
        pltpu.async_copy(src_hbm.at[s], idx_s, gsem[0])
        pltpu.async_copy(dst_hbm.at[s], idx_d, gsem[1])
        pltpu.async_copy(zeros_hbm.at[pl.ds(s * RPS, RPS)],
                         acc.at[pl.ds(s * RPS, RPS)], gsem[2])
        pltpu.make_async_copy(src_hbm.at[s], idx_s, gsem[0]).wait()
        pltpu.make_async_copy(dst_hbm.at[s], idx_d, gsem[1]).wait()
        pltpu.make_async_copy(zeros_hbm.at[pl.ds(s * RPS, RPS)],
                              acc.at[pl.ds(s * RPS, RPS)], gsem[2]).wait()
        plsc.subcore_barrier()
        table = table_hbm.at[c]

        for b in range(nb):
            pltpu.async_copy(table.at[idx_s.at[b]], rows[b], gsem[b])

        def step(i, carry):
            j = i * nb
            for b in range(nb):
                pltpu.make_async_copy(table.at[idx_s.at[j + b]], rows[b],
                                      gsem[b]).wait()
                pltpu.async_copy(rows[b], acc.at[idx_d.at[j + b]], ssem[b],
                                 add=True)
            for b in range(nb):
                pltpu.make_async_copy(rows[b], acc.at[idx_d.at[j + b]],
                                      ssem[b]).wait()
                pltpu.async_copy(table.at[idx_s.at[j + nb + b]], rows[b],
                                 gsem[b])
            return carry

        lax.fori_loop(0, NCH // nb - 1, step, 0)
        last = NCH - nb
        for b in range(nb):
            pltpu.make_async_copy(table.at[idx_s.at[last + b]], rows[b],
                                  gsem[b]).wait()
            pltpu.sync_copy(rows[b], acc.at[idx_d.at[last + b]], add=True)
        plsc.subcore_barrier()
        pltpu.sync_copy(acc.at[pl.ds(s * RPS, RPS)],
                        out_hbm.at[pl.ds(s * RPS, RPS),
                                   pl.ds(c * width, width)])

    return agg


_agg128 = _make_agg(DD // 2, NB)
_agg64 = _make_agg(CC // 2, 8)


def _dinv(degp_blk):
    deg = degp_blk[:, 0:1] + degp_blk[:, 16:17] + 1.0
    return lax.rsqrt(deg)


def _tc1_body(x_ref, wa, wb, wc, wd, degp_ref, u_ref):
    dinv = _dinv(degp_ref[...])
    W = jnp.concatenate([wa[...], wb[...], wc[...], wd[...]], axis=1)
    U = dinv * jnp.dot(x_ref[...], W,
                       precision=lax.Precision.HIGHEST,
                       preferred_element_type=jnp.float32)
    u_ref[0] = U[:, 0:DD // 2]
    u_ref[1] = U[:, DD // 2:DD]


def _tc1(x, W1a, W1b, W2a, W2b, degp):
    return pl.pallas_call(
        _tc1_body,
        grid=(GRID,),
        in_specs=[
            pl.BlockSpec((RB, DD), lambda i: (i, 0)),
            pl.BlockSpec((DD, HH), lambda i: (0, 0)),
            pl.BlockSpec((DD, HH), lambda i: (0, 0)),
            pl.BlockSpec((DD, HH), lambda i: (0, 0)),
            pl.BlockSpec((DD, HH), lambda i: (0, 0)),
            pl.BlockSpec((RB, DD), lambda i: (i, 0)),
        ],
        out_specs=pl.BlockSpec((NC, RB, DD // 2), lambda i: (0, i, 0)),
        out_shape=jax.ShapeDtypeStruct((NC, NN, DD // 2), jnp.float32),
    )(x, W1a, W1b, W2a, W2b, degp)


def _tc2_body(sp_ref, u_ref, degp_ref, b1a, b1b, b2a, b2b, wc1, wc2,
              h1_ref, h2_ref, h3_ref, h4_ref, x1_ref, x2_ref, u2_ref):
    dinv = _dinv(degp_ref[...])
    bstack = jnp.concatenate([b1a[...], b1b[...], b2a[...], b2b[...]], axis=1)
    U = jnp.concatenate([u_ref[0], u_ref[1]], axis=1)
    Y = dinv * (sp_ref[...] + U) + bstack
    Yact = jnp.maximum(Y, 0.0)
    h1_ref[...] = Yact[:, 0:HH]
    h2_ref[...] = Yact[:, HH:2 * HH]
    h3_ref[...] = Yact[:, 2 * HH:3 * HH]
    h4_ref[...] = Yact[:, 3 * HH:4 * HH]
    x1 = Yact[:, 0:HH] + Yact[:, HH:2 * HH]
    x2 = Yact[:, 2 * HH:3 * HH] + Yact[:, 3 * HH:4 * HH]
    x1_ref[...] = x1
    x2_ref[...] = x2
    U2 = dinv * jnp.dot(x1 + x2, wc1[...] + wc2[...],
                        precision=lax.Precision.HIGHEST,
                        preferred_element_type=jnp.float32)
    u2_ref[0] = U2[:, 0:CC // 2]
    u2_ref[1] = U2[:, CC // 2:CC]


def _tc2(Sp, U, degp, b1a, b1b, b2a, b2b, Wc1, Wc2):
    hspec = pl.BlockSpec((RB, HH), lambda i: (i, 0))
    hshape = jax.ShapeDtypeStruct((NN, HH), jnp.float32)
    return pl.pallas_call(
        _tc2_body,
        grid=(GRID,),
        in_specs=[
            pl.BlockSpec((RB, DD), lambda i: (i, 0)),
            pl.BlockSpec((NC, RB, DD // 2), lambda i: (0, i, 0)),
            pl.BlockSpec((RB, DD), lambda i: (i, 0)),
            pl.BlockSpec((1, HH), lambda i: (0, 0)),
            pl.BlockSpec((1, HH), lambda i: (0, 0)),
            pl.BlockSpec((1, HH), lambda i: (0, 0)),
            pl.BlockSpec((1, HH), lambda i: (0, 0)),
            pl.BlockSpec((HH, CC), lambda i: (0, 0)),
            pl.BlockSpec((HH, CC), lambda i: (0, 0)),
        ],
        out_specs=[hspec, hspec, hspec, hspec, hspec, hspec,
                   pl.BlockSpec((NC, RB, CC // 2), lambda i: (0, i, 0))],
        out_shape=[hshape, hshape, hshape, hshape, hshape, hshape,
                   jax.ShapeDtypeStruct((NC, NN, CC // 2), jnp.float32)],
    )(Sp, U, degp, b1a, b1b, b2a, b2b, Wc1, Wc2)


def _tc3_body(s2p_ref, u2_ref, degp_ref, bc1, bc2, out_ref):
    dinv = _dinv(degp_ref[...])
    U2 = jnp.concatenate([u2_ref[0], u2_ref[1]], axis=1)
    ctot = dinv * (s2p_ref[:, 0:CC] + U2) + (bc1[...] + bc2[...])
    m = jnp.max(ctot, axis=1, keepdims=True)
    lse = m + jnp.log(jnp.sum(jnp.exp(ctot - m), axis=1, keepdims=True))
    out_ref[...] = ctot - lse


def _tc3(S2p, U2, degp, bc1, bc2):
    return pl.pallas_call(
        _tc3_body,
        grid=(GRID,),
        in_specs=[
            pl.BlockSpec((RB, DD), lambda i: (i, 0)),
            pl.BlockSpec((NC, RB, CC // 2), lambda i: (0, i, 0)),
            pl.BlockSpec((RB, DD), lambda i: (i, 0)),
            pl.BlockSpec((1, CC), lambda i: (0, 0)),
            pl.BlockSpec((1, CC), lambda i: (0, 0)),
        ],
        out_specs=pl.BlockSpec((RB, CC), lambda i: (i, 0)),
        out_shape=jax.ShapeDtypeStruct((NN, CC), jnp.float32),
    )(S2p, U2, degp, bc1, bc2)


def kernel(x, edge_index, train_mask,
           W1a, b1a, W1b, b1b, W2a, b2a, W2b, b2b, Wc1, bc1, Wc2, bc2):
    src0 = edge_index[0]
    dst0 = edge_index[1]
    pad = EPAD - EE
    ar = jnp.arange(pad, dtype=jnp.int32)
    pad_src = (ar * 997) % NN
    pad_dst = NN + (ar % NTRASH)
    src_p = jnp.concatenate([src0, pad_src])
    dst_p = jnp.concatenate([dst0, pad_dst])
    src3 = src_p.reshape(NS, NCH, CH)
    dst3 = dst_p.reshape(NS, NCH, CH)
    dst3w = dst_p.reshape(NW, NCHD, CH)

    ones16 = jnp.ones((CH, 16), jnp.float32)
    zeros16 = jnp.zeros((NACC, 16), jnp.float32)
    zeros64 = jnp.zeros((NACC, DD // 2), jnp.float32)
    zeros32 = jnp.zeros((NACC, CC // 2), jnp.float32)

    degp = _deg_kernel(dst3w, ones16, zeros16)
    U = _tc1(x, W1a, W1b, W2a, W2b, degp)
    Sp = _agg128(src3, dst3, U, zeros64)
    h1, h2, h3, h4, x1, x2, U2 = _tc2(
        Sp, U, degp,
        b1a.reshape(1, HH), b1b.reshape(1, HH),
        b2a.reshape(1, HH), b2b.reshape(1, HH), Wc1, Wc2)
    S2p = _agg64(src3, dst3, U2, zeros32)
    out = _tc3(S2p, U2, degp, bc1.reshape(1, CC), bc2.reshape(1, CC))
    return (out, h1, h2, h3, h4, x1, x2)
